# Initial kernel scaffold; baseline (speedup 1.0000x reference)
import jax
import jax.numpy as jnp
from jax.experimental import pallas as pl


def kernel(X, adj, states, N, fc1_W, fc1_b, at1_W, at1_b, fc2_W, fc2_b, at2_W, at2_b, gru_Wih, gru_Whh, gru_bih, gru_bhh, resI_W, resI_b, resR_W, resR_b, sir_W, sir_b):
    raise NotImplementedError("write your pallas kernel here")


# speedup vs baseline: 4.1701x; 4.1701x over previous
import functools

import jax
import jax.numpy as jnp
from jax import lax
from jax.experimental import pallas as pl
from jax.experimental.pallas import tpu as pltpu
from jax.experimental.pallas import tpu_sc as plsc

_NB = 50000
_NZ = 50048
_NP = 51200
_NH = _NP // 2
_E = 800000
_NW = 32
_EW = 25088
_EPAD = _NW * _EW
_REG = _EW + 128
_CH = 128
_F = 32
_PCH = 1792


def _part(src_p, dst_p):
    mesh = plsc.VectorSubcoreMesh(core_axis_name="c", subcore_axis_name="s")

    @functools.partial(
        pl.kernel,
        mesh=mesh,
        compiler_params=pltpu.CompilerParams(needs_layout_passes=False,
                                             use_tc_tiling_on_sc=False),
        out_type=[
            jax.ShapeDtypeStruct((2, _NW, _REG), jnp.int32),
            jax.ShapeDtypeStruct((2, _NW, _REG), jnp.int32),
            jax.ShapeDtypeStruct((2, _NW, 16), jnp.int32),
        ],
        scratch_types=[
            pltpu.VMEM((_PCH,), jnp.int32),
            pltpu.VMEM((_PCH,), jnp.int32),
            pltpu.VMEM((_REG,), jnp.int32),
            pltpu.VMEM((_REG,), jnp.int32),
            pltpu.VMEM((_REG,), jnp.int32),
            pltpu.VMEM((_REG,), jnp.int32),
            pltpu.VMEM((16,), jnp.int32),
        ],
    )
    def k(src_h, dst_h, srcb_h, dstb_h, cnt_h,
          src_v, dst_v, sa_v, da_v, sb_v, db_v, cnt_v):
        c = lax.axis_index("c")
        s = lax.axis_index("s")
        wid = s * 2 + c

        def outer(o, carry):
            ca, cb = carry
            base = wid * _EW + o * _PCH
            pltpu.sync_copy(src_h.at[pl.ds(base, _PCH)], src_v)
            pltpu.sync_copy(dst_h.at[pl.ds(base, _PCH)], dst_v)
            for g in range(_PCH // 16):
                si = src_v[pl.ds(g * 16, 16)]
                di = dst_v[pl.ds(g * 16, 16)]
                ma = di < _NH
                plsc.store_compressed(sa_v.at[pl.ds(ca, 16)], si, mask=ma)
                plsc.store_compressed(da_v.at[pl.ds(ca, 16)], di, mask=ma)
                mb = jnp.logical_not(ma)
                plsc.store_compressed(sb_v.at[pl.ds(cb, 16)], si, mask=mb)
                plsc.store_compressed(db_v.at[pl.ds(cb, 16)], di - _NH, mask=mb)
                ca = ca + plsc.all_reduce_population_count(ma)[0]
                cb = cb + plsc.all_reduce_population_count(mb)[0]
            return ca, cb

        ca, cb = lax.fori_loop(0, _EW // _PCH, outer,
                               (jnp.int32(0), jnp.int32(0)))
        dummy_src = jnp.full((16,), _NB, jnp.int32)
        dummy_dst = jnp.zeros((16,), jnp.int32)
        for t in range(8):
            sa_v[pl.ds(ca + t * 16, 16)] = dummy_src
            da_v[pl.ds(ca + t * 16, 16)] = dummy_dst
            sb_v[pl.ds(cb + t * 16, 16)] = dummy_src
            db_v[pl.ds(cb + t * 16, 16)] = dummy_dst
        pltpu.sync_copy(sa_v, srcb_h.at[0, wid])
        pltpu.sync_copy(da_v, dstb_h.at[0, wid])
        pltpu.sync_copy(sb_v, srcb_h.at[1, wid])
        pltpu.sync_copy(db_v, dstb_h.at[1, wid])
        lanes = lax.iota(jnp.int32, 16)
        cnt_v[...] = jnp.where(lanes == 0, ca, jnp.where(lanes == 1, cb, 0))
        pltpu.sync_copy(cnt_v, cnt_h.at[0, wid])
        pltpu.sync_copy(cnt_v, cnt_h.at[1, wid])

    return k(src_p, dst_p)


def _pre_body(x_ref, w_ref, b_ref, ws_ref, wd_ref, bd_ref,
              zlo_ref, zhi_ref, as_ref, ad_ref):
    i = pl.program_id(0)
    acc = jnp.zeros((128, _F), jnp.float32)
    for t in range(8):
        acc = acc + jnp.dot(x_ref[t], w_ref[t], preferred_element_type=jnp.float32)
    z = acc + b_ref[...]
    rows = i * 128 + lax.broadcasted_iota(jnp.int32, (128, 1), 0)
    z = jnp.where(rows < _NB, z, 0.0)
    zlo_ref[...] = z[:, 0:16]
    zhi_ref[...] = z[:, 16:32]
    dn = (((1,), (1,)), ((), ()))
    a_s = lax.dot_general(ws_ref[...], z, dn, preferred_element_type=jnp.float32)
    as_ref[...] = a_s.reshape(1, 1, 128)
    ad = lax.dot_general(wd_ref[...], z, dn, preferred_element_type=jnp.float32)
    mask_l = (i * 128 + lax.broadcasted_iota(jnp.int32, (1, 128), 1)) < _NB
    ad_ref[...] = jnp.where(mask_l, ad + bd_ref[0, 0], 0.0).reshape(1, 1, 128)


def _pre(xs, w1, b1, ws, wd, bd):
    grid = _NZ // 128
    return pl.pallas_call(
        _pre_body,
        grid=(grid,),
        in_specs=[
            pl.BlockSpec((8, 128, 8), lambda i: (0, i, 0)),
            pl.BlockSpec((8, 8, _F), lambda i: (0, 0, 0)),
            pl.BlockSpec((1, _F), lambda i: (0, 0)),
            pl.BlockSpec((1, _F), lambda i: (0, 0)),
            pl.BlockSpec((1, _F), lambda i: (0, 0)),
            pl.BlockSpec((1, 1), lambda i: (0, 0)),
        ],
        out_specs=[
            pl.BlockSpec((128, 16), lambda i: (i, 0)),
            pl.BlockSpec((128, 16), lambda i: (i, 0)),
            pl.BlockSpec((1, 1, 128), lambda i: (i, 0, 0)),
            pl.BlockSpec((1, 1, 128), lambda i: (i, 0, 0)),
        ],
        out_shape=[
            jax.ShapeDtypeStruct((_NZ, 16), jnp.float32),
            jax.ShapeDtypeStruct((_NZ, 16), jnp.float32),
            jax.ShapeDtypeStruct((_NZ // 128, 1, 128), jnp.float32),
            jax.ShapeDtypeStruct((_NZ // 128, 1, 128), jnp.float32),
        ],
    )(xs, w1, b1, ws, wd, bd)


def _mid_body(plo_ref, phi_ref, w_ref, b_ref, ws_ref, wd_ref, bd_ref,
              zlo_ref, zhi_ref, as_ref, ad_ref):
    i = pl.program_id(0)
    h = jnp.concatenate([plo_ref[...], phi_ref[...]], axis=1)
    h = jnp.where(h > 0, h, jnp.exp(h) - 1.0)
    z = jnp.dot(h, w_ref[...], preferred_element_type=jnp.float32) + b_ref[...]
    rows = i * 128 + lax.broadcasted_iota(jnp.int32, (128, 1), 0)
    z = jnp.where(rows < _NB, z, 0.0)
    zlo_ref[...] = z[:, 0:16]
    zhi_ref[...] = z[:, 16:32]
    dn = (((1,), (1,)), ((), ()))
    a_s = lax.dot_general(ws_ref[...], z, dn, preferred_element_type=jnp.float32)
    as_ref[...] = a_s.reshape(1, 1, 128)
    ad = lax.dot_general(wd_ref[...], z, dn, preferred_element_type=jnp.float32)
    mask_l = (i * 128 + lax.broadcasted_iota(jnp.int32, (1, 128), 1)) < _NB
    ad_ref[...] = jnp.where(mask_l, ad + bd_ref[0, 0], 0.0).reshape(1, 1, 128)


def _mid(plo, phi, w2, b2, ws, wd, bd):
    grid = _NZ // 128
    return pl.pallas_call(
        _mid_body,
        grid=(grid,),
        in_specs=[
            pl.BlockSpec((128, 16), lambda i: (i, 0)),
            pl.BlockSpec((128, 16), lambda i: (i, 0)),
            pl.BlockSpec((_F, _F), lambda i: (0, 0)),
            pl.BlockSpec((1, _F), lambda i: (0, 0)),
            pl.BlockSpec((1, _F), lambda i: (0, 0)),
            pl.BlockSpec((1, _F), lambda i: (0, 0)),
            pl.BlockSpec((1, 1), lambda i: (0, 0)),
        ],
        out_specs=[
            pl.BlockSpec((128, 16), lambda i: (i, 0)),
            pl.BlockSpec((128, 16), lambda i: (i, 0)),
            pl.BlockSpec((1, 1, 128), lambda i: (i, 0, 0)),
            pl.BlockSpec((1, 1, 128), lambda i: (i, 0, 0)),
        ],
        out_shape=[
            jax.ShapeDtypeStruct((_NZ, 16), jnp.float32),
            jax.ShapeDtypeStruct((_NZ, 16), jnp.float32),
            jax.ShapeDtypeStruct((_NZ // 128, 1, 128), jnp.float32),
            jax.ShapeDtypeStruct((_NZ // 128, 1, 128), jnp.float32),
        ],
    )(plo, phi, w2, b2, ws, wd, bd)


def _agg(zlo, zhi, a_s, a_d, srcb, dstb, cnts):
    mesh = plsc.VectorSubcoreMesh(core_axis_name="c", subcore_axis_name="s")

    @functools.partial(
        pl.kernel,
        mesh=mesh,
        compiler_params=pltpu.CompilerParams(needs_layout_passes=False,
                                             use_tc_tiling_on_sc=False),
        out_type=[
            jax.ShapeDtypeStruct((_NP, 16), jnp.float32),
            jax.ShapeDtypeStruct((_NP, 16), jnp.float32),
        ],
        scratch_types=[
            pltpu.VMEM((_NZ,), jnp.float32),
            pltpu.VMEM((_NZ,), jnp.float32),
            pltpu.VMEM((_CH, 16), jnp.float32),
            pltpu.VMEM((_CH, 16), jnp.float32),
            pltpu.VMEM((_CH,), jnp.int32),
            pltpu.VMEM((1, _CH), jnp.int32),
            pltpu.VMEM((16,), jnp.int32),
            pltpu.VMEM_SHARED((_NH, 16), jnp.float32),
            pltpu.SemaphoreType.DMA,
        ],
    )
    def k(zlo_h, zhi_h, as_h, ad_h, srcb_h, dstb_h, cnt_h, olo_h, ohi_h,
          as_v, ad_v, rows_v, zeros_v, src_v, dst_v, cnt_v, acc_s, sem):
        c = lax.axis_index("c")
        s = lax.axis_index("s")
        pltpu.sync_copy(as_h, as_v)
        pltpu.sync_copy(ad_h, ad_v)
        zero16 = jnp.zeros((16,), jnp.float32)
        for e in range(_CH):
            zeros_v[e, pl.ds(0, 16)] = zero16
        nrows = _NH // 16
        doff = c * _NH

        for f in range(2):
            z_h = zlo_h if f == 0 else zhi_h
            out_h = olo_h if f == 0 else ohi_h
            for kk in range(nrows // _CH):
                pltpu.sync_copy(zeros_v,
                                acc_s.at[pl.ds(s * nrows + kk * _CH, _CH)])
            pltpu.sync_copy(zeros_v.at[pl.ds(0, nrows % _CH)],
                            acc_s.at[pl.ds(s * nrows + (nrows // _CH) * _CH,
                                           nrows % _CH)])
            plsc.subcore_barrier()

            for r in range(2):
                w = s * 2 + r
                pltpu.sync_copy(cnt_h.at[c, w], cnt_v)
                cload = cnt_v[...]
                cnt = jnp.where(c == 0, cload[0], cload[1])
                nch = (cnt + (_CH - 1)) // _CH

                def chunk(g, carry, w=w, z_h=z_h):
                    base = g * _CH
                    pltpu.sync_copy(srcb_h.at[c, w, pl.ds(base, _CH)], src_v)
                    pltpu.sync_copy(dstb_h.at[c, w, pl.ds(base, _CH)],
                                    dst_v.at[0])
                    pltpu.async_copy(z_h.at[src_v], rows_v, sem).wait()
                    for gr in range(_CH // 16):
                        si = src_v[pl.ds(gr * 16, 16)]
                        di = dst_v[0, pl.ds(gr * 16, 16)]
                        a = (plsc.load_gather(as_v, [si])
                             + plsc.load_gather(ad_v, [di + doff]))
                        a = jnp.maximum(a, a * 0.01)
                        for j in range(16):
                            e = gr * 16 + j
                            rows_v[e, pl.ds(0, 16)] = (
                                rows_v[e, pl.ds(0, 16)] * a[j])
                    pltpu.sync_copy(rows_v, acc_s.at[dst_v.at[0]], add=True)
                    return carry

                lax.fori_loop(0, nch, chunk, 0)

            plsc.subcore_barrier()
            pltpu.sync_copy(acc_s.at[pl.ds(s * nrows, nrows)],
                            out_h.at[pl.ds(c * _NH + s * nrows, nrows)])

    return k(zlo, zhi, a_s, a_d, srcb, dstb, cnts)


def _post_body(qlo_ref, qhi_ref, lx_ref, st_ref, n_ref, wih_ref, bih_ref,
               bhh_ref, wc_ref, wcI_ref, wcR_ref, bc_ref,
               sw_ref, swI_ref, swR_ref, sb_ref,
               pred_ref, phy_ref):
    h = jnp.concatenate([qlo_ref[...], qhi_ref[...]], axis=1)
    h = jnp.where(h > 0, h, jnp.exp(h) - 1.0)
    gx = jnp.dot(h, wih_ref[...], preferred_element_type=jnp.float32) + bih_ref[...]
    xr = gx[:, 0:32]
    xz = gx[:, 32:64]
    xn = gx[:, 64:96]
    br = bhh_ref[:, 0:32]
    bz = bhh_ref[:, 32:64]
    bn = bhh_ref[:, 64:96]
    r = jax.nn.sigmoid(xr + br)
    zg = jax.nn.sigmoid(xz + bz)
    ng = jnp.tanh(xn + r * bn)
    h_out = (1.0 - zg) * ng
    ldI = lx_ref[:, 1:2]
    ldR = lx_ref[:, 2:3]
    pred = (jnp.dot(h_out, wc_ref[...], preferred_element_type=jnp.float32)
            + ldI * wcI_ref[...] + ldR * wcR_ref[...] + bc_ref[...])
    pred_ref[...] = pred
    ab = (jnp.dot(h_out, sw_ref[...], preferred_element_type=jnp.float32)
          + ldI * swI_ref[...] + ldR * swR_ref[...] + sb_ref[...])
    alpha = jax.nn.sigmoid(ab[:, 0:1])
    beta = jax.nn.sigmoid(ab[:, 1:2])
    ns = n_ref[...]
    cur_i = st_ref[:, 0:1]
    cur_r = st_ref[:, 1:2]
    cols = []
    for _ in range(4):
        cur_s = ns - cur_i - cur_r
        d_i = alpha * cur_i * (cur_s / ns) - beta * cur_i
        d_r = beta * cur_i
        cols.append(d_i)
        cols.append(d_r)
        cur_i = cur_i + d_i
        cur_r = cur_r + d_r
    phy_ref[...] = jnp.concatenate(cols, axis=1)


def _post(qlo, qhi, lastx, states, n, wih, bih, bhh, wc, wcI, wcR, bc,
          sw, swI, swR, sb):
    blk = 400
    grid = _NB // blk
    return pl.pallas_call(
        _post_body,
        grid=(grid,),
        in_specs=[
            pl.BlockSpec((blk, 16), lambda i: (i, 0)),
            pl.BlockSpec((blk, 16), lambda i: (i, 0)),
            pl.BlockSpec((blk, 8), lambda i: (i, 0)),
            pl.BlockSpec((blk, 2), lambda i: (i, 0)),
            pl.BlockSpec((blk, 1), lambda i: (i, 0)),
            pl.BlockSpec((_F, 96), lambda i: (0, 0)),
            pl.BlockSpec((1, 96), lambda i: (0, 0)),
            pl.BlockSpec((1, 96), lambda i: (0, 0)),
            pl.BlockSpec((_F, 8), lambda i: (0, 0)),
            pl.BlockSpec((1, 8), lambda i: (0, 0)),
            pl.BlockSpec((1, 8), lambda i: (0, 0)),
            pl.BlockSpec((1, 8), lambda i: (0, 0)),
            pl.BlockSpec((_F, 2), lambda i: (0, 0)),
            pl.BlockSpec((1, 2), lambda i: (0, 0)),
            pl.BlockSpec((1, 2), lambda i: (0, 0)),
            pl.BlockSpec((1, 2), lambda i: (0, 0)),
        ],
        out_specs=[
            pl.BlockSpec((blk, 8), lambda i: (i, 0)),
            pl.BlockSpec((blk, 8), lambda i: (i, 0)),
        ],
        out_shape=[
            jax.ShapeDtypeStruct((_NB, 8), jnp.float32),
            jax.ShapeDtypeStruct((_NB, 8), jnp.float32),
        ],
    )(qlo, qhi, lastx, states, n, wih, bih, bhh, wc, wcI, wcR, bc,
      sw, swI, swR, sb)


def kernel(X, adj, states, N, fc1_W, fc1_b, at1_W, at1_b, fc2_W, fc2_b,
           at2_W, at2_b, gru_Wih, gru_Whh, gru_bih, gru_bhh,
           resI_W, resI_b, resR_W, resR_b, sir_W, sir_b):
    xs = X[0]
    pad = _EPAD - _E
    src_p = jnp.concatenate([adj[0], jnp.full((pad,), _NB, jnp.int32)])
    dst_p = jnp.concatenate([adj[1], jnp.zeros((pad,), jnp.int32)])
    srcb, dstb, cnts = _part(src_p, dst_p)

    z1lo, z1hi, a1s, a1d = _pre(xs, fc1_W.reshape(8, 8, _F),
                                fc1_b.reshape(1, _F),
                                at1_W[:_F].reshape(1, _F),
                                at1_W[_F:].reshape(1, _F), at1_b.reshape(1, 1))
    p1lo, p1hi = _agg(z1lo, z1hi, a1s.reshape(_NZ), a1d.reshape(_NZ),
                      srcb, dstb, cnts)
    z2lo, z2hi, a2s, a2d = _mid(p1lo, p1hi, fc2_W, fc2_b.reshape(1, _F),
                                at2_W[:_F].reshape(1, _F),
                                at2_W[_F:].reshape(1, _F), at2_b.reshape(1, 1))
    qlo, qhi = _agg(z2lo, z2hi, a2s.reshape(_NZ), a2d.reshape(_NZ),
                    srcb, dstb, cnts)

    wc = jnp.stack([resI_W, resR_W], axis=-1).reshape(34, 8)
    bc = jnp.stack([resI_b, resR_b], axis=-1).reshape(1, 8)
    pred8, phy8 = _post(qlo, qhi, X[0, -1], states, N,
                        gru_Wih, gru_bih.reshape(1, 96), gru_bhh.reshape(1, 96),
                        wc[:_F], wc[_F:_F + 1], wc[_F + 1:], bc,
                        sir_W[:_F], sir_W[_F:_F + 1], sir_W[_F + 1:],
                        sir_b.reshape(1, 2))
    return pred8.reshape(_NB, 4, 2), phy8.reshape(_NB, 4, 2)


# speedup vs baseline: 5.4761x; 1.3132x over previous
import functools

import jax
import jax.numpy as jnp
from jax import lax
from jax.experimental import pallas as pl
from jax.experimental.pallas import tpu as pltpu
from jax.experimental.pallas import tpu_sc as plsc

_NB = 50000
_NZ = 50048
_NP = 50176
_NH = _NP // 2
_E = 800000
_NW = 32
_EW = 25088
_EPAD = _NW * _EW
_REG = _EW + 512
_CH = 128
_F = 32
_PCH = 1792


def _part(src_p, dst_p):
    mesh = plsc.VectorSubcoreMesh(core_axis_name="c", subcore_axis_name="s")

    @functools.partial(
        pl.kernel,
        mesh=mesh,
        compiler_params=pltpu.CompilerParams(needs_layout_passes=False,
                                             use_tc_tiling_on_sc=False),
        out_type=[
            jax.ShapeDtypeStruct((2, _NW, _REG), jnp.int32),
            jax.ShapeDtypeStruct((2, _NW, _REG), jnp.int32),
            jax.ShapeDtypeStruct((2, _NW, 16), jnp.int32),
        ],
        scratch_types=[
            pltpu.VMEM((_PCH,), jnp.int32),
            pltpu.VMEM((_PCH,), jnp.int32),
            pltpu.VMEM((_REG,), jnp.int32),
            pltpu.VMEM((_REG,), jnp.int32),
            pltpu.VMEM((_REG,), jnp.int32),
            pltpu.VMEM((_REG,), jnp.int32),
            pltpu.VMEM((16,), jnp.int32),
        ],
    )
    def k(src_h, dst_h, srcb_h, dstb_h, cnt_h,
          src_v, dst_v, sa_v, da_v, sb_v, db_v, cnt_v):
        c = lax.axis_index("c")
        s = lax.axis_index("s")
        wid = s * 2 + c

        def outer(o, carry):
            ca, cb = carry
            base = wid * _EW + o * _PCH
            pltpu.sync_copy(src_h.at[pl.ds(base, _PCH)], src_v)
            pltpu.sync_copy(dst_h.at[pl.ds(base, _PCH)], dst_v)
            for g in range(_PCH // 16):
                si = src_v[pl.ds(g * 16, 16)]
                di = dst_v[pl.ds(g * 16, 16)]
                ma = di < _NH
                plsc.store_compressed(sa_v.at[pl.ds(ca, 16)], si, mask=ma)
                plsc.store_compressed(da_v.at[pl.ds(ca, 16)], di, mask=ma)
                mb = jnp.logical_not(ma)
                plsc.store_compressed(sb_v.at[pl.ds(cb, 16)], si, mask=mb)
                plsc.store_compressed(db_v.at[pl.ds(cb, 16)], di - _NH, mask=mb)
                ca = ca + plsc.all_reduce_population_count(ma)[0]
                cb = cb + plsc.all_reduce_population_count(mb)[0]
            return ca, cb

        ca, cb = lax.fori_loop(0, _EW // _PCH, outer,
                               (jnp.int32(0), jnp.int32(0)))
        dummy_src = jnp.full((16,), _NB, jnp.int32)
        dummy_dst = jnp.zeros((16,), jnp.int32)
        for t in range(32):
            sa_v[pl.ds(ca + t * 16, 16)] = dummy_src
            da_v[pl.ds(ca + t * 16, 16)] = dummy_dst
            sb_v[pl.ds(cb + t * 16, 16)] = dummy_src
            db_v[pl.ds(cb + t * 16, 16)] = dummy_dst
        pltpu.sync_copy(sa_v, srcb_h.at[0, wid])
        pltpu.sync_copy(da_v, dstb_h.at[0, wid])
        pltpu.sync_copy(sb_v, srcb_h.at[1, wid])
        pltpu.sync_copy(db_v, dstb_h.at[1, wid])
        lanes = lax.iota(jnp.int32, 16)
        cnt_v[...] = jnp.where(lanes == 0, ca, jnp.where(lanes == 1, cb, 0))
        pltpu.sync_copy(cnt_v, cnt_h.at[0, wid])
        pltpu.sync_copy(cnt_v, cnt_h.at[1, wid])

    return k(src_p, dst_p)


def _pre_body(x_ref, w_ref, b_ref, ws_ref, wd_ref, bd_ref,
              zlo_ref, zhi_ref, as_ref, ad_ref):
    i = pl.program_id(0)
    acc = jnp.zeros((128, _F), jnp.float32)
    for t in range(8):
        acc = acc + jnp.dot(x_ref[t], w_ref[t], preferred_element_type=jnp.float32)
    z = acc + b_ref[...]
    rows = i * 128 + lax.broadcasted_iota(jnp.int32, (128, 1), 0)
    z = jnp.where(rows < _NB, z, 0.0)
    zlo_ref[...] = z[:, 0:16]
    zhi_ref[...] = z[:, 16:32]
    dn = (((1,), (1,)), ((), ()))
    a_s = lax.dot_general(ws_ref[...], z, dn, preferred_element_type=jnp.float32)
    as_ref[...] = a_s.reshape(1, 1, 128)
    ad = lax.dot_general(wd_ref[...], z, dn, preferred_element_type=jnp.float32)
    mask_l = (i * 128 + lax.broadcasted_iota(jnp.int32, (1, 128), 1)) < _NB
    ad_ref[...] = jnp.where(mask_l, ad + bd_ref[0, 0], 0.0).reshape(1, 1, 128)


def _pre(xs, w1, b1, ws, wd, bd):
    grid = _NZ // 128
    return pl.pallas_call(
        _pre_body,
        grid=(grid,),
        in_specs=[
            pl.BlockSpec((8, 128, 8), lambda i: (0, i, 0)),
            pl.BlockSpec((8, 8, _F), lambda i: (0, 0, 0)),
            pl.BlockSpec((1, _F), lambda i: (0, 0)),
            pl.BlockSpec((1, _F), lambda i: (0, 0)),
            pl.BlockSpec((1, _F), lambda i: (0, 0)),
            pl.BlockSpec((1, 1), lambda i: (0, 0)),
        ],
        out_specs=[
            pl.BlockSpec((128, 16), lambda i: (i, 0)),
            pl.BlockSpec((128, 16), lambda i: (i, 0)),
            pl.BlockSpec((1, 1, 128), lambda i: (i, 0, 0)),
            pl.BlockSpec((1, 1, 128), lambda i: (i, 0, 0)),
        ],
        out_shape=[
            jax.ShapeDtypeStruct((_NZ, 16), jnp.float32),
            jax.ShapeDtypeStruct((_NZ, 16), jnp.float32),
            jax.ShapeDtypeStruct((_NZ // 128, 1, 128), jnp.float32),
            jax.ShapeDtypeStruct((_NZ // 128, 1, 128), jnp.float32),
        ],
    )(xs, w1, b1, ws, wd, bd)


def _mid_body(plo_ref, phi_ref, w_ref, b_ref, ws_ref, wd_ref, bd_ref,
              zlo_ref, zhi_ref, as_ref, ad_ref):
    i = pl.program_id(0)
    h = jnp.concatenate([plo_ref[...], phi_ref[...]], axis=1)
    h = jnp.where(h > 0, h, jnp.exp(h) - 1.0)
    z = jnp.dot(h, w_ref[...], preferred_element_type=jnp.float32) + b_ref[...]
    rows = i * 128 + lax.broadcasted_iota(jnp.int32, (128, 1), 0)
    z = jnp.where(rows < _NB, z, 0.0)
    zlo_ref[...] = z[:, 0:16]
    zhi_ref[...] = z[:, 16:32]
    dn = (((1,), (1,)), ((), ()))
    a_s = lax.dot_general(ws_ref[...], z, dn, preferred_element_type=jnp.float32)
    as_ref[...] = a_s.reshape(1, 1, 128)
    ad = lax.dot_general(wd_ref[...], z, dn, preferred_element_type=jnp.float32)
    mask_l = (i * 128 + lax.broadcasted_iota(jnp.int32, (1, 128), 1)) < _NB
    ad_ref[...] = jnp.where(mask_l, ad + bd_ref[0, 0], 0.0).reshape(1, 1, 128)


def _mid(plo, phi, w2, b2, ws, wd, bd):
    grid = _NZ // 128
    return pl.pallas_call(
        _mid_body,
        grid=(grid,),
        in_specs=[
            pl.BlockSpec((128, 16), lambda i: (i, 0)),
            pl.BlockSpec((128, 16), lambda i: (i, 0)),
            pl.BlockSpec((_F, _F), lambda i: (0, 0)),
            pl.BlockSpec((1, _F), lambda i: (0, 0)),
            pl.BlockSpec((1, _F), lambda i: (0, 0)),
            pl.BlockSpec((1, _F), lambda i: (0, 0)),
            pl.BlockSpec((1, 1), lambda i: (0, 0)),
        ],
        out_specs=[
            pl.BlockSpec((128, 16), lambda i: (i, 0)),
            pl.BlockSpec((128, 16), lambda i: (i, 0)),
            pl.BlockSpec((1, 1, 128), lambda i: (i, 0, 0)),
            pl.BlockSpec((1, 1, 128), lambda i: (i, 0, 0)),
        ],
        out_shape=[
            jax.ShapeDtypeStruct((_NZ, 16), jnp.float32),
            jax.ShapeDtypeStruct((_NZ, 16), jnp.float32),
            jax.ShapeDtypeStruct((_NZ // 128, 1, 128), jnp.float32),
            jax.ShapeDtypeStruct((_NZ // 128, 1, 128), jnp.float32),
        ],
    )(plo, phi, w2, b2, ws, wd, bd)


_NBUF = 2
_BLK = 4


def _agg(zlo, zhi, a_s, a_d, srcb, dstb4, cnts):
    mesh = plsc.VectorSubcoreMesh(core_axis_name="c", subcore_axis_name="s")

    @functools.partial(
        pl.kernel,
        mesh=mesh,
        compiler_params=pltpu.CompilerParams(needs_layout_passes=False,
                                             use_tc_tiling_on_sc=False),
        out_type=[
            jax.ShapeDtypeStruct((_NP, 16), jnp.float32),
            jax.ShapeDtypeStruct((_NP, 16), jnp.float32),
        ],
        scratch_types=[
            pltpu.VMEM((_NZ,), jnp.float32),
            pltpu.VMEM((_NZ,), jnp.float32),
            [pltpu.VMEM((_CH, 16), jnp.float32) for _ in range(_NBUF)],
            pltpu.VMEM((32, 16), jnp.float32),
            pltpu.VMEM((_BLK * _CH,), jnp.int32),
            pltpu.VMEM((_BLK, _CH), jnp.int32),
            pltpu.VMEM((16,), jnp.int32),
            pltpu.VMEM_SHARED((_NH, 16), jnp.float32),
            [pltpu.SemaphoreType.DMA for _ in range(_NBUF)],
        ],
    )
    def k(zlo_h, zhi_h, as_h, ad_h, srcb_h, dstb_h, cnt_h, olo_h, ohi_h,
          as_v, ad_v, rows, zeros_v, src_v, dst_v, cnt_v, acc_s, gsem):
        c = lax.axis_index("c")
        s = lax.axis_index("s")
        pltpu.sync_copy(as_h, as_v)
        pltpu.sync_copy(ad_h, ad_v)
        zero16 = jnp.zeros((16,), jnp.float32)
        for e in range(32):
            zeros_v[e, pl.ds(0, 16)] = zero16
        nrows = _NH // 16
        doff = c * _NH

        def compute(j, b):
            for gr in range(_CH // 16):
                si = src_v[pl.ds(j * _CH + gr * 16, 16)]
                di = dst_v[j, pl.ds(gr * 16, 16)]
                a = (plsc.load_gather(as_v, [si])
                     + plsc.load_gather(ad_v, [di + doff]))
                a = jnp.maximum(a, a * 0.01)
                for jj in range(16):
                    e = gr * 16 + jj
                    rows[b][e, pl.ds(0, 16)] = rows[b][e, pl.ds(0, 16)] * a[jj]

        for f in range(2):
            z_h = zlo_h if f == 0 else zhi_h
            out_h = olo_h if f == 0 else ohi_h
            for kk in range(nrows // 32):
                pltpu.sync_copy(zeros_v,
                                acc_s.at[pl.ds(s * nrows + kk * 32, 32)])
            plsc.subcore_barrier()

            for r in range(2):
                w = s * 2 + r
                pltpu.sync_copy(cnt_h.at[c, w], cnt_v)
                cload = cnt_v[...]
                cnt = jnp.where(c == 0, cload[0], cload[1])
                nblk = (cnt + (_BLK * _CH - 1)) // (_BLK * _CH)

                def block(bk, carry, w=w, z_h=z_h):
                    pltpu.sync_copy(
                        srcb_h.at[c, w, pl.ds(bk * (_BLK * _CH), _BLK * _CH)],
                        src_v)
                    pltpu.sync_copy(dstb_h.at[c, w, pl.ds(bk * _BLK, _BLK)],
                                    dst_v)
                    gh = [None] * _BLK
                    for j in range(2):
                        gh[j] = pltpu.async_copy(
                            z_h.at[src_v.at[pl.ds(j * _CH, _CH)]],
                            rows[j % _NBUF], gsem[j % _NBUF])
                    for j in range(_BLK):
                        b = j % _NBUF
                        gh[j].wait()
                        compute(j, b)
                        pltpu.sync_copy(rows[b], acc_s.at[dst_v.at[j]],
                                        add=True)
                        nj = j + 2
                        if nj < _BLK:
                            gh[nj] = pltpu.async_copy(
                                z_h.at[src_v.at[pl.ds(nj * _CH, _CH)]],
                                rows[nj % _NBUF], gsem[nj % _NBUF])
                    return carry

                lax.fori_loop(0, nblk, block, 0)

            plsc.subcore_barrier()
            pltpu.sync_copy(acc_s.at[pl.ds(s * nrows, nrows)],
                            out_h.at[pl.ds(c * _NH + s * nrows, nrows)])

    return k(zlo, zhi, a_s, a_d, srcb, dstb4, cnts)


def _post_body(qlo_ref, qhi_ref, lx_ref, st_ref, n_ref, wih_ref, bih_ref,
               bhh_ref, wc_ref, wcI_ref, wcR_ref, bc_ref,
               sw_ref, swI_ref, swR_ref, sb_ref,
               pred_ref, phy_ref):
    h = jnp.concatenate([qlo_ref[...], qhi_ref[...]], axis=1)
    h = jnp.where(h > 0, h, jnp.exp(h) - 1.0)
    gx = jnp.dot(h, wih_ref[...], preferred_element_type=jnp.float32) + bih_ref[...]
    xr = gx[:, 0:32]
    xz = gx[:, 32:64]
    xn = gx[:, 64:96]
    br = bhh_ref[:, 0:32]
    bz = bhh_ref[:, 32:64]
    bn = bhh_ref[:, 64:96]
    r = jax.nn.sigmoid(xr + br)
    zg = jax.nn.sigmoid(xz + bz)
    ng = jnp.tanh(xn + r * bn)
    h_out = (1.0 - zg) * ng
    ldI = lx_ref[:, 1:2]
    ldR = lx_ref[:, 2:3]
    pred = (jnp.dot(h_out, wc_ref[...], preferred_element_type=jnp.float32)
            + ldI * wcI_ref[...] + ldR * wcR_ref[...] + bc_ref[...])
    pred_ref[...] = pred
    ab = (jnp.dot(h_out, sw_ref[...], preferred_element_type=jnp.float32)
          + ldI * swI_ref[...] + ldR * swR_ref[...] + sb_ref[...])
    alpha = jax.nn.sigmoid(ab[:, 0:1])
    beta = jax.nn.sigmoid(ab[:, 1:2])
    ns = n_ref[...]
    cur_i = st_ref[:, 0:1]
    cur_r = st_ref[:, 1:2]
    cols = []
    for _ in range(4):
        cur_s = ns - cur_i - cur_r
        d_i = alpha * cur_i * (cur_s / ns) - beta * cur_i
        d_r = beta * cur_i
        cols.append(d_i)
        cols.append(d_r)
        cur_i = cur_i + d_i
        cur_r = cur_r + d_r
    phy_ref[...] = jnp.concatenate(cols, axis=1)


def _post(qlo, qhi, lastx, states, n, wih, bih, bhh, wc, wcI, wcR, bc,
          sw, swI, swR, sb):
    blk = 400
    grid = _NB // blk
    return pl.pallas_call(
        _post_body,
        grid=(grid,),
        in_specs=[
            pl.BlockSpec((blk, 16), lambda i: (i, 0)),
            pl.BlockSpec((blk, 16), lambda i: (i, 0)),
            pl.BlockSpec((blk, 8), lambda i: (i, 0)),
            pl.BlockSpec((blk, 2), lambda i: (i, 0)),
            pl.BlockSpec((blk, 1), lambda i: (i, 0)),
            pl.BlockSpec((_F, 96), lambda i: (0, 0)),
            pl.BlockSpec((1, 96), lambda i: (0, 0)),
            pl.BlockSpec((1, 96), lambda i: (0, 0)),
            pl.BlockSpec((_F, 8), lambda i: (0, 0)),
            pl.BlockSpec((1, 8), lambda i: (0, 0)),
            pl.BlockSpec((1, 8), lambda i: (0, 0)),
            pl.BlockSpec((1, 8), lambda i: (0, 0)),
            pl.BlockSpec((_F, 2), lambda i: (0, 0)),
            pl.BlockSpec((1, 2), lambda i: (0, 0)),
            pl.BlockSpec((1, 2), lambda i: (0, 0)),
            pl.BlockSpec((1, 2), lambda i: (0, 0)),
        ],
        out_specs=[
            pl.BlockSpec((blk, 8), lambda i: (i, 0)),
            pl.BlockSpec((blk, 8), lambda i: (i, 0)),
        ],
        out_shape=[
            jax.ShapeDtypeStruct((_NB, 8), jnp.float32),
            jax.ShapeDtypeStruct((_NB, 8), jnp.float32),
        ],
    )(qlo, qhi, lastx, states, n, wih, bih, bhh, wc, wcI, wcR, bc,
      sw, swI, swR, sb)


def kernel(X, adj, states, N, fc1_W, fc1_b, at1_W, at1_b, fc2_W, fc2_b,
           at2_W, at2_b, gru_Wih, gru_Whh, gru_bih, gru_bhh,
           resI_W, resI_b, resR_W, resR_b, sir_W, sir_b):
    xs = X[0]
    pad = _EPAD - _E
    src_p = jnp.concatenate([adj[0], jnp.full((pad,), _NB, jnp.int32)])
    dst_p = jnp.concatenate([adj[1], jnp.zeros((pad,), jnp.int32)])
    srcb, dstb, cnts = _part(src_p, dst_p)

    z1lo, z1hi, a1s, a1d = _pre(xs, fc1_W.reshape(8, 8, _F),
                                fc1_b.reshape(1, _F),
                                at1_W[:_F].reshape(1, _F),
                                at1_W[_F:].reshape(1, _F), at1_b.reshape(1, 1))
    dstb4 = dstb.reshape(2, _NW, _REG // _CH, _CH)
    p1lo, p1hi = _agg(z1lo, z1hi, a1s.reshape(_NZ), a1d.reshape(_NZ),
                      srcb, dstb4, cnts)
    z2lo, z2hi, a2s, a2d = _mid(p1lo, p1hi, fc2_W, fc2_b.reshape(1, _F),
                                at2_W[:_F].reshape(1, _F),
                                at2_W[_F:].reshape(1, _F), at2_b.reshape(1, 1))
    qlo, qhi = _agg(z2lo, z2hi, a2s.reshape(_NZ), a2d.reshape(_NZ),
                    srcb, dstb4, cnts)

    wc = jnp.stack([resI_W, resR_W], axis=-1).reshape(34, 8)
    bc = jnp.stack([resI_b, resR_b], axis=-1).reshape(1, 8)
    pred8, phy8 = _post(qlo, qhi, X[0, -1], states, N,
                        gru_Wih, gru_bih.reshape(1, 96), gru_bhh.reshape(1, 96),
                        wc[:_F], wc[_F:_F + 1], wc[_F + 1:], bc,
                        sir_W[:_F], sir_W[_F:_F + 1], sir_W[_F + 1:],
                        sir_b.reshape(1, 2))
    return pred8.reshape(_NB, 4, 2), phy8.reshape(_NB, 4, 2)
